# baseline (speedup 1.0000x reference)
import functools

import jax
import jax.numpy as jnp
from jax import lax
from jax.experimental import pallas as pl
from jax.experimental.pallas import tpu as pltpu
from jax.experimental.pallas import tpu_sc as plsc

_CODEBOOK = 8192
_D = 32
_BETA = 0.2
_ROWS_PER_BLOCK = 1024


def _argmin_body(x_ref, emb_ref, idx_ref, loss_ref, esq_ref, iota_ref):
    step = pl.program_id(0)

    @pl.when(step == 0)
    def _():
        emb = emb_ref[...]
        esq_ref[0, :] = jnp.sum(emb * emb, axis=1)
        iota_ref[...] = lax.broadcasted_iota(
            jnp.int32, (1, _CODEBOOK), 1).astype(jnp.float32)
        loss_ref[...] = jnp.zeros((1, 1), jnp.float32)

    x = x_ref[...]
    xsq = jnp.sum(x * x, axis=1, keepdims=True)
    neg2mm = lax.dot_general(x * -2.0, emb_ref[...],
                             (((1,), (1,)), ((), ())),
                             preferred_element_type=jnp.float32)
    d = (xsq + esq_ref[...]) + neg2mm
    mind = jnp.min(d, axis=1, keepdims=True)
    idx_ref[...] = jnp.argmin(d, axis=1, keepdims=True).astype(jnp.int32)

    loss_ref[...] += jnp.sum(mind)

    @pl.when(step == pl.num_programs(0) - 1)
    def _():
        n_elems = pl.num_programs(0) * _ROWS_PER_BLOCK * _D
        loss_ref[...] = loss_ref[...] * ((1.0 + _BETA) / n_elems)


def _indexes_and_loss(x_flat, embedding):
    n = x_flat.shape[0]
    grid = n // _ROWS_PER_BLOCK
    idx3, loss = pl.pallas_call(
        _argmin_body,
        grid=(grid,),
        in_specs=[
            pl.BlockSpec((_ROWS_PER_BLOCK, _D), lambda i: (i, 0)),
            pl.BlockSpec((_CODEBOOK, _D), lambda i: (0, 0)),
        ],
        out_specs=[
            pl.BlockSpec((_ROWS_PER_BLOCK, 1), lambda i: (i, 0)),
            pl.BlockSpec((1, 1), lambda i: (0, 0)),
        ],
        out_shape=[
            jax.ShapeDtypeStruct((n, 1), jnp.int32),
            jax.ShapeDtypeStruct((1, 1), jnp.float32),
        ],
        scratch_shapes=[pltpu.VMEM((1, _CODEBOOK), jnp.float32),
                        pltpu.VMEM((1, _CODEBOOK), jnp.float32)],
    )(x_flat, embedding)
    return idx3.reshape(n), loss.reshape(())


def _gather_rows(embedding, idx_flat):
    info = plsc.get_sparse_core_info()
    nc, ns = info.num_cores, info.num_subcores
    nw = nc * ns
    b = idx_flat.shape[0]
    b_per_w = b // nw
    chunk = 128
    k = b_per_w // chunk
    idx2 = idx_flat.reshape(b // chunk, chunk)
    mesh = plsc.VectorSubcoreMesh(core_axis_name="c", subcore_axis_name="s")

    @functools.partial(
        pl.kernel, mesh=mesh,
        out_type=jax.ShapeDtypeStruct((b, _D), jnp.float32),
        compiler_params=pltpu.CompilerParams(use_tc_tiling_on_sc=False),
        scratch_types=[
            pltpu.VMEM((k, chunk), jnp.int32),
            pltpu.VMEM((k, chunk, _D), jnp.float32),
            pltpu.SemaphoreType.DMA,
        ],
    )
    def gk(table_hbm, idx_hbm, out_hbm, idx_v, rows_v, sem):
        wid = lax.axis_index("s") * nc + lax.axis_index("c")
        pltpu.sync_copy(idx_hbm.at[pl.ds(wid * k, k)], idx_v)
        for j in range(k):
            pltpu.async_copy(table_hbm.at[idx_v.at[j]], rows_v.at[j], sem).wait()
        for j in range(k):
            pltpu.sync_copy(rows_v.at[j],
                            out_hbm.at[pl.ds((wid * k + j) * chunk, chunk)])

    return gk(embedding, idx2)


def kernel(x, embedding):
    bsz, seq, d = x.shape
    x_flat = x.reshape(bsz * seq, d)
    idx_flat, loss = _indexes_and_loss(x_flat, embedding)
    values = _gather_rows(embedding, idx_flat)
    return values.reshape(bsz, seq, d), idx_flat.reshape(bsz, seq), loss


# speedup vs baseline: 1.0152x; 1.0152x over previous
import functools

import jax
import jax.numpy as jnp
from jax import lax
from jax.experimental import pallas as pl
from jax.experimental.pallas import tpu as pltpu
from jax.experimental.pallas import tpu_sc as plsc

_CODEBOOK = 8192
_D = 32
_BETA = 0.2
_ROWS_PER_BLOCK = 512


def _argmin_body(x_ref, emb_ref, idx_ref, loss_ref, esq_ref, iota_ref):
    step = pl.program_id(0)

    @pl.when(step == 0)
    def _():
        emb = emb_ref[...]
        esq_ref[0, :] = jnp.sum(emb * emb, axis=1)
        iota_ref[...] = lax.broadcasted_iota(
            jnp.int32, (1, _CODEBOOK), 1).astype(jnp.float32)
        loss_ref[...] = jnp.zeros((1, 1), jnp.float32)

    x = x_ref[...]
    xsq = jnp.sum(x * x, axis=1, keepdims=True)
    neg2mm = lax.dot_general(x * -2.0, emb_ref[...],
                             (((1,), (1,)), ((), ())),
                             preferred_element_type=jnp.float32)
    d = (xsq + esq_ref[...]) + neg2mm
    mind = jnp.min(d, axis=1, keepdims=True)
    idx_ref[...] = jnp.argmin(d, axis=1, keepdims=True).astype(jnp.int32)

    loss_ref[...] += jnp.sum(mind)

    @pl.when(step == pl.num_programs(0) - 1)
    def _():
        n_elems = pl.num_programs(0) * _ROWS_PER_BLOCK * _D
        loss_ref[...] = loss_ref[...] * ((1.0 + _BETA) / n_elems)


def _indexes_and_loss(x_flat, embedding):
    n = x_flat.shape[0]
    grid = n // _ROWS_PER_BLOCK
    idx3, loss = pl.pallas_call(
        _argmin_body,
        grid=(grid,),
        in_specs=[
            pl.BlockSpec((_ROWS_PER_BLOCK, _D), lambda i: (i, 0)),
            pl.BlockSpec((_CODEBOOK, _D), lambda i: (0, 0)),
        ],
        out_specs=[
            pl.BlockSpec((_ROWS_PER_BLOCK, 1), lambda i: (i, 0)),
            pl.BlockSpec((1, 1), lambda i: (0, 0)),
        ],
        out_shape=[
            jax.ShapeDtypeStruct((n, 1), jnp.int32),
            jax.ShapeDtypeStruct((1, 1), jnp.float32),
        ],
        scratch_shapes=[pltpu.VMEM((1, _CODEBOOK), jnp.float32),
                        pltpu.VMEM((1, _CODEBOOK), jnp.float32)],
    )(x_flat, embedding)
    return idx3.reshape(n), loss.reshape(())


def _gather_rows(embedding, idx_flat):
    info = plsc.get_sparse_core_info()
    nc, ns = info.num_cores, info.num_subcores
    nw = nc * ns
    b = idx_flat.shape[0]
    b_per_w = b // nw
    chunk = 128
    k = b_per_w // chunk
    idx2 = idx_flat.reshape(b // chunk, chunk)
    mesh = plsc.VectorSubcoreMesh(core_axis_name="c", subcore_axis_name="s")

    @functools.partial(
        pl.kernel, mesh=mesh,
        out_type=jax.ShapeDtypeStruct((b, _D), jnp.float32),
        compiler_params=pltpu.CompilerParams(use_tc_tiling_on_sc=False),
        scratch_types=[
            pltpu.VMEM((k, chunk), jnp.int32),
            pltpu.VMEM((k, chunk, _D), jnp.float32),
            pltpu.SemaphoreType.DMA,
            pltpu.SemaphoreType.DMA,
        ],
    )
    def gk(table_hbm, idx_hbm, out_hbm, idx_v, rows_v, sem, sem_out):
        wid = lax.axis_index("s") * nc + lax.axis_index("c")
        pltpu.sync_copy(idx_hbm.at[pl.ds(wid * k, k)], idx_v)
        gathers = [
            pltpu.async_copy(table_hbm.at[idx_v.at[j]], rows_v.at[j], sem)
            for j in range(k)
        ]
        writes = []
        for j in range(k):
            gathers[j].wait()
            writes.append(pltpu.async_copy(
                rows_v.at[j],
                out_hbm.at[pl.ds((wid * k + j) * chunk, chunk)], sem_out))
        for w in writes:
            w.wait()

    return gk(embedding, idx2)


def kernel(x, embedding):
    bsz, seq, d = x.shape
    x_flat = x.reshape(bsz * seq, d)
    idx_flat, loss = _indexes_and_loss(x_flat, embedding)
    values = _gather_rows(embedding, idx_flat)
    return values.reshape(bsz, seq, d), idx_flat.reshape(bsz, seq), loss
